# baseline (speedup 1.0000x reference)
import functools

import jax
import jax.numpy as jnp
from jax import lax
from jax.experimental import pallas as pl
from jax.experimental.pallas import tpu as pltpu

DIM = 32
N_EMBED = 1024
BT = 8192


def _vq_block(x_ref, w_ref, q_ref, diff_ref, ind_ref):
    x = x_ref[...]
    w = w_ref[...]
    e2 = jnp.sum(w * w, axis=0)
    sT = lax.dot_general(w, x, (((0,), (1,)), ((), ())),
                         preferred_element_type=jnp.float32)
    sT = sT - 0.5 * e2[:, None]
    ind = jnp.argmax(sT, axis=0).astype(jnp.int32)
    onehot = (lax.broadcasted_iota(jnp.int32, (N_EMBED, BT), 0)
              == ind[None, :]).astype(jnp.float32)
    qT = lax.dot_general(w, onehot, (((1,), (0,)), ((), ())),
                         preferred_element_type=jnp.float32)
    q = qT.T
    q_ref[...] = x + (q - x)
    diff_ref[...] = (q - x) ** 2
    ind_ref[...] = ind[:, None]


def kernel(inputs, embed):
    n_tokens = inputs.shape[0]
    grid = (n_tokens // BT,)
    q, diff, ind = pl.pallas_call(
        _vq_block,
        grid=grid,
        in_specs=[
            pl.BlockSpec((BT, DIM), lambda i: (i, 0)),
            pl.BlockSpec((DIM, N_EMBED), lambda i: (0, 0)),
        ],
        out_specs=[
            pl.BlockSpec((BT, DIM), lambda i: (i, 0)),
            pl.BlockSpec((BT, DIM), lambda i: (i, 0)),
            pl.BlockSpec((BT, 1), lambda i: (i, 0)),
        ],
        out_shape=[
            jax.ShapeDtypeStruct((n_tokens, DIM), jnp.float32),
            jax.ShapeDtypeStruct((n_tokens, DIM), jnp.float32),
            jax.ShapeDtypeStruct((n_tokens, 1), jnp.int32),
        ],
    )(inputs, embed)
    return (q, diff.reshape(n_tokens, DIM, 1), ind)


# speedup vs baseline: 1.6376x; 1.6376x over previous
import jax
import jax.numpy as jnp
from jax import lax
from jax.experimental import pallas as pl

DIM = 32
N_EMBED = 1024
BT = 8192


def _vq_block(xt_ref, w_ref, qt_ref, difft_ref, ind_ref):
    xt = xt_ref[...]
    w = w_ref[...]
    e2 = jnp.sum(w * w, axis=0)
    sT = lax.dot_general(w, xt, (((0,), (0,)), ((), ())),
                         preferred_element_type=jnp.float32)
    sT = sT - 0.5 * e2[:, None]
    ind = jnp.argmax(sT, axis=0).astype(jnp.int32)
    onehot = (lax.broadcasted_iota(jnp.int32, (N_EMBED, BT), 0)
              == ind[None, :]).astype(jnp.float32)
    qt = lax.dot_general(w, onehot, (((1,), (0,)), ((), ())),
                         preferred_element_type=jnp.float32)
    qt_ref[...] = xt + (qt - xt)
    difft_ref[...] = (qt - xt) ** 2
    ind_ref[...] = ind


def kernel(inputs, embed):
    n_tokens = inputs.shape[0]
    grid = (n_tokens // BT,)
    qt, difft, ind = pl.pallas_call(
        _vq_block,
        grid=grid,
        in_specs=[
            pl.BlockSpec((DIM, BT), lambda i: (0, i)),
            pl.BlockSpec((DIM, N_EMBED), lambda i: (0, 0)),
        ],
        out_specs=[
            pl.BlockSpec((DIM, BT), lambda i: (0, i)),
            pl.BlockSpec((DIM, BT), lambda i: (0, i)),
            pl.BlockSpec((BT,), lambda i: (i,)),
        ],
        out_shape=[
            jax.ShapeDtypeStruct((DIM, n_tokens), jnp.float32),
            jax.ShapeDtypeStruct((DIM, n_tokens), jnp.float32),
            jax.ShapeDtypeStruct((n_tokens,), jnp.int32),
        ],
    )(inputs.T, embed)
    return (qt.T, difft.T.reshape(n_tokens, DIM, 1), ind.reshape(n_tokens, 1))
